# baseline (speedup 1.0000x reference)
import functools

import jax
import jax.numpy as jnp
from jax import lax
from jax.experimental import pallas as pl
from jax.experimental.pallas import tpu as pltpu
from jax.experimental.pallas import tpu_sc as plsc

D_FEAT = 256
HIDDEN = 256
OUT_DIM = 256


def _pack_halves(y):
    bc = lax.bitcast_convert_type
    half = y.shape[-1] // 2
    u_lo = bc(y[:, :half], jnp.int32) + jnp.int32(0x8000)
    u_hi = bc(y[:, half:], jnp.int32) + jnp.int32(0x8000)
    return lax.bitwise_or(
        lax.bitwise_and(u_hi, jnp.int32(-65536)),
        lax.shift_right_logical(u_lo, 16),
    )


def _unpack_halves(y32):
    bc = lax.bitcast_convert_type
    lo = bc(lax.shift_left(y32, 16), jnp.float32)
    hi = bc(lax.bitwise_and(y32, jnp.int32(-65536)), jnp.float32)
    return jnp.concatenate([lo, hi], axis=1)


_PROJ_BLK = 2000


def _proj_body(x_ref, ws_ref, wd_ref, ys_ref, yd_ref, x_out_ref):
    xb = x_ref[...]
    ys_ref[...] = _pack_halves(
        jnp.dot(xb, ws_ref[...], preferred_element_type=jnp.float32)
    )
    yd_ref[...] = _pack_halves(
        jnp.dot(xb, wd_ref[...], preferred_element_type=jnp.float32)
    )
    x_out_ref[...] = xb


def _project_nodes(x, w1):
    n = x.shape[0]
    grid = n // _PROJ_BLK
    return pl.pallas_call(
        _proj_body,
        grid=(grid,),
        in_specs=[
            pl.BlockSpec((_PROJ_BLK, D_FEAT), lambda i: (i, 0)),
            pl.BlockSpec((D_FEAT, HIDDEN), lambda i: (0, 0)),
            pl.BlockSpec((D_FEAT, HIDDEN), lambda i: (1, 0)),
        ],
        out_specs=[
            pl.BlockSpec((_PROJ_BLK, HIDDEN // 2), lambda i: (i, 0)),
            pl.BlockSpec((_PROJ_BLK, HIDDEN // 2), lambda i: (i, 0)),
            pl.BlockSpec((_PROJ_BLK, D_FEAT), lambda i: (i, 0)),
        ],
        out_shape=[
            jax.ShapeDtypeStruct((n, HIDDEN // 2), jnp.int32),
            jax.ShapeDtypeStruct((n, HIDDEN // 2), jnp.int32),
            jax.ShapeDtypeStruct((n, D_FEAT), jnp.float32),
        ],
    )(x, w1, w1)


_CH = 128


def _make_gather_add(n_edges, call_off):
    info = plsc.get_sparse_core_info()
    nw = info.num_cores * info.num_subcores
    e_per_w = n_edges // nw
    assert call_off % 8 == 0
    assert e_per_w * nw == n_edges and e_per_w % 8 == 0
    n_full = e_per_w // _CH
    if n_full % 2:
        n_full -= 1
    n_tail = -((n_full * _CH - e_per_w) // _CH)
    tail_offs = [e_per_w - i * _CH for i in range(n_tail, 0, -1)]
    assert all(off % 8 == 0 and off >= n_full * _CH - _CH for off in tail_offs)

    mesh = plsc.VectorSubcoreMesh(core_axis_name="c", subcore_axis_name="s")

    @functools.partial(
        pl.kernel,
        mesh=mesh,
        out_type=jax.ShapeDtypeStruct((n_edges, HIDDEN // 2), jnp.int32),
        scratch_types=[
            pltpu.VMEM((e_per_w,), jnp.int32),
            pltpu.VMEM((e_per_w,), jnp.int32),
            pltpu.VMEM((2, _CH, HIDDEN // 2), jnp.int32),
            pltpu.VMEM((2, _CH, HIDDEN // 2), jnp.int32),
            pltpu.SemaphoreType.DMA,
            pltpu.SemaphoreType.DMA,
            pltpu.SemaphoreType.DMA,
            pltpu.SemaphoreType.DMA,
            pltpu.SemaphoreType.DMA,
            pltpu.SemaphoreType.DMA,
        ],
    )
    def gather_add(
        ys, yd, src, dst, out,
        idx_s, idx_d, buf_a, buf_b,
        sem_a0, sem_a1, sem_b0, sem_b1, sem_w0, sem_w1,
    ):
        wid = lax.axis_index("s") * info.num_cores + lax.axis_index("c")
        base0 = wid * e_per_w
        sems_a = (sem_a0, sem_a1)
        sems_b = (sem_b0, sem_b1)
        sems_w = (sem_w0, sem_w1)

        pltpu.sync_copy(src.at[pl.ds(call_off + base0, e_per_w)], idx_s)
        pltpu.sync_copy(dst.at[pl.ds(call_off + base0, e_per_w)], idx_d)

        def gathers(off, slot):
            pltpu.async_copy(
                ys.at[idx_s.at[pl.ds(off, _CH)]], buf_a.at[slot], sems_a[slot]
            )
            pltpu.async_copy(
                yd.at[idx_d.at[pl.ds(off, _CH)]], buf_b.at[slot], sems_b[slot]
            )

        def wait_gathers(off, slot):
            pltpu.make_async_copy(
                ys.at[idx_s.at[pl.ds(off, _CH)]], buf_a.at[slot], sems_a[slot]
            ).wait()
            pltpu.make_async_copy(
                yd.at[idx_d.at[pl.ds(off, _CH)]], buf_b.at[slot], sems_b[slot]
            ).wait()

        def writeback(off, slot):
            pltpu.async_copy(
                buf_a.at[slot], out.at[pl.ds(base0 + off, _CH)], sems_w[slot]
            )

        def wait_writeback(off, slot):
            pltpu.make_async_copy(
                buf_a.at[slot], out.at[pl.ds(base0 + off, _CH)], sems_w[slot]
            ).wait()

        mask_hi = jnp.int32(-65536)
        half = jnp.int32(0x8000)

        def bf16_pair_add(a, b):
            bc = lax.bitcast_convert_type
            a_lo = bc(lax.shift_left(a, 16), jnp.float32)
            b_lo = bc(lax.shift_left(b, 16), jnp.float32)
            u_lo = bc(a_lo + b_lo, jnp.int32) + half
            s_hi = bc(bc(a, jnp.float32) + bc(b, jnp.float32), jnp.int32)
            return lax.bitwise_or(
                lax.bitwise_and(s_hi, mask_hi),
                lax.shift_right_logical(u_lo, 16),
            )

        def add_rows(slot):
            @plsc.parallel_loop(0, _CH, unroll=2)
            def _(i):
                for j in range(HIDDEN // 32):
                    sl = pl.ds(j * 16, 16)
                    buf_a[slot, i, sl] = bf16_pair_add(
                        buf_a[slot, i, sl], buf_b[slot, i, sl]
                    )

        gathers(0, 0)
        gathers(_CH, 1)

        def step(c, slot):
            off = c * _CH
            wait_gathers(off, slot)
            add_rows(slot)

            @pl.when(c >= 2)
            def _():
                wait_writeback(off - 2 * _CH, slot)

            writeback(off, slot)

            @pl.when(c + 2 < n_full)
            def _():
                gathers(off + 2 * _CH, slot)

        def pair_body(p, carry):
            step(2 * p, 0)
            step(2 * p + 1, 1)
            return carry

        lax.fori_loop(0, n_full // 2, pair_body, 0)

        wait_writeback((n_full - 2) * _CH, 0)
        wait_writeback((n_full - 1) * _CH, 1)

        for t, off in enumerate(tail_offs):
            slot = t % 2
            gathers(off, slot)
            wait_gathers(off, slot)
            add_rows(slot)
            pltpu.sync_copy(buf_a.at[slot], out.at[pl.ds(base0 + off, _CH)])

    return gather_add


_MLP_BLK = 3200


def _mlp_body(g_ref, attr_t_ref, we_ref, b1_ref, w2_ref, b2_ref, out_ref):
    h_attr = lax.dot_general(
        attr_t_ref[...],
        we_ref[...],
        (((0,), (0,)), ((), ())),
        preferred_element_type=jnp.float32,
    )
    h = _unpack_halves(g_ref[...]) + h_attr + b1_ref[...]
    h = jnp.maximum(h, 0.0).astype(jnp.bfloat16)
    out_ref[...] = (
        jnp.dot(h, w2_ref[...], preferred_element_type=jnp.float32) + b2_ref[...]
    )


def _mlp_body_aliased(g_ref, attr_ref, w1_ref, b1_ref, w2_ref, b2_ref, h_ref, out_ref):
    del h_ref
    _mlp_body(g_ref, attr_ref, w1_ref, b1_ref, w2_ref, b2_ref, out_ref)


def _edge_mlp(g, attr_t, w1, b1, w2, b2, *, out_rows, blk_off, h_alias=None):
    e = g.shape[0]
    d_edge = attr_t.shape[0]
    grid = e // _MLP_BLK
    in_specs = [
        pl.BlockSpec((_MLP_BLK, HIDDEN // 2), lambda i: (i, 0)),
        pl.BlockSpec((d_edge, _MLP_BLK), lambda i: (0, i + blk_off)),
        pl.BlockSpec((d_edge, HIDDEN), lambda i: (2 * D_FEAT // 16, 0)),
        pl.BlockSpec((1, HIDDEN), lambda i: (0, 0)),
        pl.BlockSpec((HIDDEN, OUT_DIM), lambda i: (0, 0)),
        pl.BlockSpec((1, OUT_DIM), lambda i: (0, 0)),
    ]
    args = [g, attr_t, w1, b1, w2, b2]
    body = _mlp_body
    aliases = {}
    if h_alias is not None:
        in_specs.append(pl.BlockSpec(memory_space=pl.ANY))
        args.append(h_alias)
        body = _mlp_body_aliased
        aliases = {6: 0}
    return pl.pallas_call(
        body,
        grid=(grid,),
        in_specs=in_specs,
        out_specs=pl.BlockSpec((_MLP_BLK, OUT_DIM), lambda i: (i + blk_off, 0)),
        out_shape=jax.ShapeDtypeStruct((out_rows, OUT_DIM), jnp.float32),
        input_output_aliases=aliases,
    )(*args)


def kernel(x, edge_index, edge_attr, W1, b1, W2, b2):
    src = edge_index[0].astype(jnp.int32)
    dst = edge_index[1].astype(jnp.int32)

    ys32, yd32, x_out = _project_nodes(x, W1)
    e = edge_attr.shape[0]
    b1r = b1.reshape(1, -1)
    b2r = b2.reshape(1, -1)
    w2b = W2.astype(jnp.bfloat16)

    s1 = 76800
    assert s1 % _MLP_BLK == 0 and (e - s1) % _MLP_BLK == 0
    attr_t = edge_attr.T
    ga = _make_gather_add(s1, 0)(ys32, yd32, src, dst)
    gb = _make_gather_add(e - s1, s1)(ys32, yd32, src, dst)
    h1 = _edge_mlp(
        ga, attr_t, W1, b1r, w2b, b2r,
        out_rows=e, blk_off=0,
    )
    h_e = _edge_mlp(
        gb, attr_t, W1, b1r, w2b, b2r,
        out_rows=e, blk_off=s1 // _MLP_BLK, h_alias=h1,
    )
    return (x_out, edge_index, h_e)


# speedup vs baseline: 1.0372x; 1.0372x over previous
import functools

import jax
import jax.numpy as jnp
from jax import lax
from jax.experimental import pallas as pl
from jax.experimental.pallas import tpu as pltpu
from jax.experimental.pallas import tpu_sc as plsc

D_FEAT = 256
HIDDEN = 256
OUT_DIM = 256


def _pack_halves(y):
    bc = lax.bitcast_convert_type
    half = y.shape[-1] // 2
    u_lo = bc(y[:, :half], jnp.int32) + jnp.int32(0x8000)
    u_hi = bc(y[:, half:], jnp.int32) + jnp.int32(0x8000)
    return lax.bitwise_or(
        lax.bitwise_and(u_hi, jnp.int32(-65536)),
        lax.shift_right_logical(u_lo, 16),
    )


def _unpack_halves(y32):
    bc = lax.bitcast_convert_type
    lo = bc(lax.shift_left(y32, 16), jnp.float32)
    hi = bc(lax.bitwise_and(y32, jnp.int32(-65536)), jnp.float32)
    return jnp.concatenate([lo, hi], axis=1)


_PROJ_BLK = 2000


def _proj_body(x_ref, ws_ref, wd_ref, ys_ref, yd_ref, x_out_ref):
    xb = x_ref[...]
    ys_ref[...] = _pack_halves(
        jnp.dot(xb, ws_ref[...], preferred_element_type=jnp.float32)
    )
    yd_ref[...] = _pack_halves(
        jnp.dot(xb, wd_ref[...], preferred_element_type=jnp.float32)
    )
    x_out_ref[...] = xb


def _project_nodes(x, w1):
    n = x.shape[0]
    grid = n // _PROJ_BLK
    return pl.pallas_call(
        _proj_body,
        grid=(grid,),
        in_specs=[
            pl.BlockSpec((_PROJ_BLK, D_FEAT), lambda i: (i, 0)),
            pl.BlockSpec((D_FEAT, HIDDEN), lambda i: (0, 0)),
            pl.BlockSpec((D_FEAT, HIDDEN), lambda i: (1, 0)),
        ],
        out_specs=[
            pl.BlockSpec((_PROJ_BLK, HIDDEN // 2), lambda i: (i, 0)),
            pl.BlockSpec((_PROJ_BLK, HIDDEN // 2), lambda i: (i, 0)),
            pl.BlockSpec((_PROJ_BLK, D_FEAT), lambda i: (i, 0)),
        ],
        out_shape=[
            jax.ShapeDtypeStruct((n, HIDDEN // 2), jnp.int32),
            jax.ShapeDtypeStruct((n, HIDDEN // 2), jnp.int32),
            jax.ShapeDtypeStruct((n, D_FEAT), jnp.float32),
        ],
    )(x, w1, w1)


_CH = 128


def _make_gather_add(n_edges, call_off):
    info = plsc.get_sparse_core_info()
    nw = info.num_cores * info.num_subcores
    e_per_w = n_edges // nw
    assert call_off % 8 == 0
    assert e_per_w * nw == n_edges and e_per_w % 8 == 0
    n_full = e_per_w // _CH
    if n_full % 2:
        n_full -= 1
    n_tail = -((n_full * _CH - e_per_w) // _CH)
    tail_offs = [e_per_w - i * _CH for i in range(n_tail, 0, -1)]
    assert all(off % 8 == 0 and off >= n_full * _CH - _CH for off in tail_offs)

    mesh = plsc.VectorSubcoreMesh(core_axis_name="c", subcore_axis_name="s")

    @functools.partial(
        pl.kernel,
        mesh=mesh,
        out_type=jax.ShapeDtypeStruct((n_edges, HIDDEN // 2), jnp.int32),
        scratch_types=[
            pltpu.VMEM((e_per_w,), jnp.int32),
            pltpu.VMEM((e_per_w,), jnp.int32),
            pltpu.VMEM((2, _CH, HIDDEN // 2), jnp.int32),
            pltpu.VMEM((2, _CH, HIDDEN // 2), jnp.int32),
            pltpu.SemaphoreType.DMA,
            pltpu.SemaphoreType.DMA,
            pltpu.SemaphoreType.DMA,
            pltpu.SemaphoreType.DMA,
            pltpu.SemaphoreType.DMA,
            pltpu.SemaphoreType.DMA,
        ],
    )
    def gather_add(
        ys, yd, src, dst, out,
        idx_s, idx_d, buf_a, buf_b,
        sem_a0, sem_a1, sem_b0, sem_b1, sem_w0, sem_w1,
    ):
        wid = lax.axis_index("s") * info.num_cores + lax.axis_index("c")
        base0 = wid * e_per_w
        sems_a = (sem_a0, sem_a1)
        sems_b = (sem_b0, sem_b1)
        sems_w = (sem_w0, sem_w1)

        pltpu.sync_copy(src.at[pl.ds(call_off + base0, e_per_w)], idx_s)
        pltpu.sync_copy(dst.at[pl.ds(call_off + base0, e_per_w)], idx_d)

        def gathers(off, slot):
            pltpu.async_copy(
                ys.at[idx_s.at[pl.ds(off, _CH)]], buf_a.at[slot], sems_a[slot]
            )
            pltpu.async_copy(
                yd.at[idx_d.at[pl.ds(off, _CH)]], buf_b.at[slot], sems_b[slot]
            )

        def wait_gathers(off, slot):
            pltpu.make_async_copy(
                ys.at[idx_s.at[pl.ds(off, _CH)]], buf_a.at[slot], sems_a[slot]
            ).wait()
            pltpu.make_async_copy(
                yd.at[idx_d.at[pl.ds(off, _CH)]], buf_b.at[slot], sems_b[slot]
            ).wait()

        def writeback(off, slot):
            pltpu.async_copy(
                buf_a.at[slot], out.at[pl.ds(base0 + off, _CH)], sems_w[slot]
            )

        def wait_writeback(off, slot):
            pltpu.make_async_copy(
                buf_a.at[slot], out.at[pl.ds(base0 + off, _CH)], sems_w[slot]
            ).wait()

        mask_hi = jnp.int32(-65536)
        half = jnp.int32(0x8000)

        def bf16_pair_add(a, b):
            bc = lax.bitcast_convert_type
            a_lo = bc(lax.shift_left(a, 16), jnp.float32)
            b_lo = bc(lax.shift_left(b, 16), jnp.float32)
            u_lo = bc(a_lo + b_lo, jnp.int32) + half
            s_hi = bc(bc(a, jnp.float32) + bc(b, jnp.float32), jnp.int32)
            return lax.bitwise_or(
                lax.bitwise_and(s_hi, mask_hi),
                lax.shift_right_logical(u_lo, 16),
            )

        def add_rows(slot):
            @plsc.parallel_loop(0, _CH, unroll=2)
            def _(i):
                for j in range(HIDDEN // 32):
                    sl = pl.ds(j * 16, 16)
                    buf_a[slot, i, sl] = bf16_pair_add(
                        buf_a[slot, i, sl], buf_b[slot, i, sl]
                    )

        gathers(0, 0)
        gathers(_CH, 1)

        def step(c, slot):
            off = c * _CH
            wait_gathers(off, slot)
            add_rows(slot)

            @pl.when(c >= 2)
            def _():
                wait_writeback(off - 2 * _CH, slot)

            writeback(off, slot)

            @pl.when(c + 2 < n_full)
            def _():
                gathers(off + 2 * _CH, slot)

        def pair_body(p, carry):
            step(2 * p, 0)
            step(2 * p + 1, 1)
            return carry

        lax.fori_loop(0, n_full // 2, pair_body, 0)

        wait_writeback((n_full - 2) * _CH, 0)
        wait_writeback((n_full - 1) * _CH, 1)

        for t, off in enumerate(tail_offs):
            slot = t % 2
            gathers(off, slot)
            wait_gathers(off, slot)
            add_rows(slot)
            pltpu.sync_copy(buf_a.at[slot], out.at[pl.ds(base0 + off, _CH)])

    return gather_add


_MLP_BLK = 6400


def _mlp_body(g_ref, attr_t_ref, we_ref, b1_ref, w2_ref, b2_ref, out_ref):
    h_attr = lax.dot_general(
        attr_t_ref[...],
        we_ref[...],
        (((0,), (0,)), ((), ())),
        preferred_element_type=jnp.float32,
    )
    h = _unpack_halves(g_ref[...]) + h_attr + b1_ref[...]
    h = jnp.maximum(h, 0.0).astype(jnp.bfloat16)
    out_ref[...] = (
        jnp.dot(h, w2_ref[...], preferred_element_type=jnp.float32) + b2_ref[...]
    )


def _mlp_body_aliased(g_ref, attr_ref, w1_ref, b1_ref, w2_ref, b2_ref, h_ref, out_ref):
    del h_ref
    _mlp_body(g_ref, attr_ref, w1_ref, b1_ref, w2_ref, b2_ref, out_ref)


def _edge_mlp(g, attr_t, w1, b1, w2, b2, *, out_rows, blk_off, h_alias=None):
    e = g.shape[0]
    d_edge = attr_t.shape[0]
    grid = e // _MLP_BLK
    in_specs = [
        pl.BlockSpec((_MLP_BLK, HIDDEN // 2), lambda i: (i, 0)),
        pl.BlockSpec((d_edge, _MLP_BLK), lambda i: (0, i + blk_off)),
        pl.BlockSpec((d_edge, HIDDEN), lambda i: (2 * D_FEAT // 16, 0)),
        pl.BlockSpec((1, HIDDEN), lambda i: (0, 0)),
        pl.BlockSpec((HIDDEN, OUT_DIM), lambda i: (0, 0)),
        pl.BlockSpec((1, OUT_DIM), lambda i: (0, 0)),
    ]
    args = [g, attr_t, w1, b1, w2, b2]
    body = _mlp_body
    aliases = {}
    if h_alias is not None:
        in_specs.append(pl.BlockSpec(memory_space=pl.ANY))
        args.append(h_alias)
        body = _mlp_body_aliased
        aliases = {6: 0}
    return pl.pallas_call(
        body,
        grid=(grid,),
        in_specs=in_specs,
        out_specs=pl.BlockSpec((_MLP_BLK, OUT_DIM), lambda i: (i + blk_off, 0)),
        out_shape=jax.ShapeDtypeStruct((out_rows, OUT_DIM), jnp.float32),
        input_output_aliases=aliases,
    )(*args)


def kernel(x, edge_index, edge_attr, W1, b1, W2, b2):
    src = edge_index[0].astype(jnp.int32)
    dst = edge_index[1].astype(jnp.int32)

    ys32, yd32, x_out = _project_nodes(x, W1)
    e = edge_attr.shape[0]
    b1r = b1.reshape(1, -1)
    b2r = b2.reshape(1, -1)
    w2b = W2.astype(jnp.bfloat16)

    s1 = 83200
    assert s1 % _MLP_BLK == 0 and (e - s1) % _MLP_BLK == 0
    attr_t = edge_attr.T
    ga = _make_gather_add(s1, 0)(ys32, yd32, src, dst)
    gb = _make_gather_add(e - s1, s1)(ys32, yd32, src, dst)
    h1 = _edge_mlp(
        ga, attr_t, W1, b1r, w2b, b2r,
        out_rows=e, blk_off=0,
    )
    h_e = _edge_mlp(
        gb, attr_t, W1, b1r, w2b, b2r,
        out_rows=e, blk_off=s1 // _MLP_BLK, h_alias=h1,
    )
    return (x_out, edge_index, h_e)


# speedup vs baseline: 1.0662x; 1.0280x over previous
import functools

import jax
import jax.numpy as jnp
from jax import lax
from jax.experimental import pallas as pl
from jax.experimental.pallas import tpu as pltpu
from jax.experimental.pallas import tpu_sc as plsc

D_FEAT = 256
HIDDEN = 256
OUT_DIM = 256


def _pack_halves(y):
    bc = lax.bitcast_convert_type
    half = y.shape[-1] // 2
    u_lo = bc(y[:, :half], jnp.int32) + jnp.int32(0x8000)
    u_hi = bc(y[:, half:], jnp.int32) + jnp.int32(0x8000)
    return lax.bitwise_or(
        lax.bitwise_and(u_hi, jnp.int32(-65536)),
        lax.shift_right_logical(u_lo, 16),
    )


def _unpack_halves(y32):
    bc = lax.bitcast_convert_type
    lo = bc(lax.shift_left(y32, 16), jnp.float32)
    hi = bc(lax.bitwise_and(y32, jnp.int32(-65536)), jnp.float32)
    return jnp.concatenate([lo, hi], axis=1)


_PROJ_BLK = 2000


def _proj_body(x_ref, ws_ref, wd_ref, ys_ref, yd_ref, x_out_ref):
    xb = x_ref[...]
    ys_ref[...] = _pack_halves(
        jnp.dot(xb, ws_ref[...], preferred_element_type=jnp.float32)
    )
    yd_ref[...] = _pack_halves(
        jnp.dot(xb, wd_ref[...], preferred_element_type=jnp.float32)
    )
    x_out_ref[...] = xb


def _project_nodes(x, w1):
    n = x.shape[0]
    grid = n // _PROJ_BLK
    return pl.pallas_call(
        _proj_body,
        grid=(grid,),
        in_specs=[
            pl.BlockSpec((_PROJ_BLK, D_FEAT), lambda i: (i, 0)),
            pl.BlockSpec((D_FEAT, HIDDEN), lambda i: (0, 0)),
            pl.BlockSpec((D_FEAT, HIDDEN), lambda i: (1, 0)),
        ],
        out_specs=[
            pl.BlockSpec((_PROJ_BLK, HIDDEN // 2), lambda i: (i, 0)),
            pl.BlockSpec((_PROJ_BLK, HIDDEN // 2), lambda i: (i, 0)),
            pl.BlockSpec((_PROJ_BLK, D_FEAT), lambda i: (i, 0)),
        ],
        out_shape=[
            jax.ShapeDtypeStruct((n, HIDDEN // 2), jnp.int32),
            jax.ShapeDtypeStruct((n, HIDDEN // 2), jnp.int32),
            jax.ShapeDtypeStruct((n, D_FEAT), jnp.float32),
        ],
    )(x, w1, w1)


_CH = 128


def _make_gather_add(n_edges, call_off):
    info = plsc.get_sparse_core_info()
    nw = info.num_cores * info.num_subcores
    e_per_w = n_edges // nw
    assert call_off % 8 == 0
    assert e_per_w * nw == n_edges and e_per_w % 8 == 0
    n_full = e_per_w // _CH
    if n_full % 2:
        n_full -= 1
    n_tail = -((n_full * _CH - e_per_w) // _CH)
    tail_offs = [e_per_w - i * _CH for i in range(n_tail, 0, -1)]
    assert all(off % 8 == 0 and off >= n_full * _CH - _CH for off in tail_offs)

    mesh = plsc.VectorSubcoreMesh(core_axis_name="c", subcore_axis_name="s")

    @functools.partial(
        pl.kernel,
        mesh=mesh,
        out_type=jax.ShapeDtypeStruct((n_edges, HIDDEN // 2), jnp.int32),
        scratch_types=[
            pltpu.VMEM((e_per_w,), jnp.int32),
            pltpu.VMEM((e_per_w,), jnp.int32),
            pltpu.VMEM((2, _CH, HIDDEN // 2), jnp.int32),
            pltpu.VMEM((2, _CH, HIDDEN // 2), jnp.int32),
            pltpu.SemaphoreType.DMA,
            pltpu.SemaphoreType.DMA,
            pltpu.SemaphoreType.DMA,
            pltpu.SemaphoreType.DMA,
            pltpu.SemaphoreType.DMA,
            pltpu.SemaphoreType.DMA,
        ],
    )
    def gather_add(
        ys, yd, src, dst, out,
        idx_s, idx_d, buf_a, buf_b,
        sem_a0, sem_a1, sem_b0, sem_b1, sem_w0, sem_w1,
    ):
        wid = lax.axis_index("s") * info.num_cores + lax.axis_index("c")
        base0 = wid * e_per_w
        sems_a = (sem_a0, sem_a1)
        sems_b = (sem_b0, sem_b1)
        sems_w = (sem_w0, sem_w1)

        pltpu.sync_copy(src.at[pl.ds(call_off + base0, e_per_w)], idx_s)
        pltpu.sync_copy(dst.at[pl.ds(call_off + base0, e_per_w)], idx_d)

        def gathers(off, slot):
            pltpu.async_copy(
                ys.at[idx_s.at[pl.ds(off, _CH)]], buf_a.at[slot], sems_a[slot]
            )
            pltpu.async_copy(
                yd.at[idx_d.at[pl.ds(off, _CH)]], buf_b.at[slot], sems_b[slot]
            )

        def wait_gathers(off, slot):
            pltpu.make_async_copy(
                ys.at[idx_s.at[pl.ds(off, _CH)]], buf_a.at[slot], sems_a[slot]
            ).wait()
            pltpu.make_async_copy(
                yd.at[idx_d.at[pl.ds(off, _CH)]], buf_b.at[slot], sems_b[slot]
            ).wait()

        def writeback(off, slot):
            pltpu.async_copy(
                buf_a.at[slot], out.at[pl.ds(base0 + off, _CH)], sems_w[slot]
            )

        def wait_writeback(off, slot):
            pltpu.make_async_copy(
                buf_a.at[slot], out.at[pl.ds(base0 + off, _CH)], sems_w[slot]
            ).wait()

        mask_hi = jnp.int32(-65536)
        half = jnp.int32(0x8000)

        def bf16_pair_add(a, b):
            bc = lax.bitcast_convert_type
            a_lo = bc(lax.shift_left(a, 16), jnp.float32)
            b_lo = bc(lax.shift_left(b, 16), jnp.float32)
            u_lo = bc(a_lo + b_lo, jnp.int32) + half
            s_hi = bc(bc(a, jnp.float32) + bc(b, jnp.float32), jnp.int32)
            return lax.bitwise_or(
                lax.bitwise_and(s_hi, mask_hi),
                lax.shift_right_logical(u_lo, 16),
            )

        def add_rows(slot):
            @plsc.parallel_loop(0, _CH, unroll=2)
            def _(i):
                for j in range(HIDDEN // 32):
                    sl = pl.ds(j * 16, 16)
                    buf_a[slot, i, sl] = bf16_pair_add(
                        buf_a[slot, i, sl], buf_b[slot, i, sl]
                    )

        gathers(0, 0)
        gathers(_CH, 1)

        def step(c, slot):
            off = c * _CH
            wait_gathers(off, slot)
            add_rows(slot)

            @pl.when(c >= 2)
            def _():
                wait_writeback(off - 2 * _CH, slot)

            writeback(off, slot)

            @pl.when(c + 2 < n_full)
            def _():
                gathers(off + 2 * _CH, slot)

        def pair_body(p, carry):
            step(2 * p, 0)
            step(2 * p + 1, 1)
            return carry

        lax.fori_loop(0, n_full // 2, pair_body, 0)

        wait_writeback((n_full - 2) * _CH, 0)
        wait_writeback((n_full - 1) * _CH, 1)

        for t, off in enumerate(tail_offs):
            slot = t % 2
            gathers(off, slot)
            wait_gathers(off, slot)
            add_rows(slot)
            pltpu.sync_copy(buf_a.at[slot], out.at[pl.ds(base0 + off, _CH)])

    return gather_add


_MLP_BLK = 16000


def _mlp_body(g_ref, attr_t_ref, we_ref, b1_ref, w2_ref, b2_ref, out_ref):
    h_attr = lax.dot_general(
        attr_t_ref[...],
        we_ref[...],
        (((0,), (0,)), ((), ())),
        preferred_element_type=jnp.float32,
    )
    h = _unpack_halves(g_ref[...]) + h_attr + b1_ref[...]
    h = jnp.maximum(h, 0.0).astype(jnp.bfloat16)
    out_ref[...] = (
        jnp.dot(h, w2_ref[...], preferred_element_type=jnp.float32) + b2_ref[...]
    )


def _mlp_body_aliased(g_ref, attr_ref, w1_ref, b1_ref, w2_ref, b2_ref, h_ref, out_ref):
    del h_ref
    _mlp_body(g_ref, attr_ref, w1_ref, b1_ref, w2_ref, b2_ref, out_ref)


def _edge_mlp(g, attr_t, w1, b1, w2, b2, *, out_rows, blk_off, h_alias=None):
    e = g.shape[0]
    d_edge = attr_t.shape[0]
    grid = e // _MLP_BLK
    in_specs = [
        pl.BlockSpec((_MLP_BLK, HIDDEN // 2), lambda i: (i, 0)),
        pl.BlockSpec((d_edge, _MLP_BLK), lambda i: (0, i + blk_off)),
        pl.BlockSpec((d_edge, HIDDEN), lambda i: (2 * D_FEAT // 16, 0)),
        pl.BlockSpec((1, HIDDEN), lambda i: (0, 0)),
        pl.BlockSpec((HIDDEN, OUT_DIM), lambda i: (0, 0)),
        pl.BlockSpec((1, OUT_DIM), lambda i: (0, 0)),
    ]
    args = [g, attr_t, w1, b1, w2, b2]
    body = _mlp_body
    aliases = {}
    if h_alias is not None:
        in_specs.append(pl.BlockSpec(memory_space=pl.ANY))
        args.append(h_alias)
        body = _mlp_body_aliased
        aliases = {6: 0}
    return pl.pallas_call(
        body,
        grid=(grid,),
        in_specs=in_specs,
        out_specs=pl.BlockSpec((_MLP_BLK, OUT_DIM), lambda i: (i + blk_off, 0)),
        out_shape=jax.ShapeDtypeStruct((out_rows, OUT_DIM), jnp.float32),
        input_output_aliases=aliases,
    )(*args)


def kernel(x, edge_index, edge_attr, W1, b1, W2, b2):
    src = edge_index[0].astype(jnp.int32)
    dst = edge_index[1].astype(jnp.int32)

    ys32, yd32, x_out = _project_nodes(x, W1)
    e = edge_attr.shape[0]
    b1r = b1.reshape(1, -1)
    b2r = b2.reshape(1, -1)
    w2b = W2.astype(jnp.bfloat16)

    s1 = 96000
    assert s1 % _MLP_BLK == 0 and (e - s1) % _MLP_BLK == 0
    attr_t = edge_attr.T
    ga = _make_gather_add(s1, 0)(ys32, yd32, src, dst)
    gb = _make_gather_add(e - s1, s1)(ys32, yd32, src, dst)
    h1 = _edge_mlp(
        ga, attr_t, W1, b1r, w2b, b2r,
        out_rows=e, blk_off=0,
    )
    h_e = _edge_mlp(
        gb, attr_t, W1, b1r, w2b, b2r,
        out_rows=e, blk_off=s1 // _MLP_BLK, h_alias=h1,
    )
    return (x_out, edge_index, h_e)
